# Initial kernel scaffold; baseline (speedup 1.0000x reference)
import jax
import jax.numpy as jnp
from jax.experimental import pallas as pl


def kernel(idx, W1, W2, W3):
    raise NotImplementedError("write your pallas kernel here")


# speedup vs baseline: 2.9594x; 2.9594x over previous
import functools

import jax
import jax.numpy as jnp
from jax import lax
from jax.experimental import pallas as pl
from jax.experimental.pallas import tpu as pltpu
from jax.experimental.pallas import tpu_sc as plsc

BATCH, HIST = 4096, 50
N = BATCH * HIST
LANES = 128
NROWS = N // LANES
D = 64
OUT_D = 3 * D
TRES = 24.0
TSCALE = 10.0
V1, V2, V3 = 366, 24, 100000

NC, NS = 2, 16
NW = NC * NS
RPW = NROWS // NW

GRID = 25
W3B = V3 // GRID
IDXB = NROWS // GRID


def _prep_body(idx_ref, w1_ref, w2_ref, w3_ref,
               i1_ref, i2_ref, i3_ref, o1_ref, o2_ref, o3_ref):
    j = pl.program_id(0)
    x = idx_ref[...]
    t1 = (x * TRES).astype(jnp.int32)
    i1_ref[...] = lax.rem(lax.div(t1, 24), 366)
    i2_ref[...] = lax.rem(t1, 24)
    i3_ref[...] = (x / TSCALE).astype(jnp.int32)

    def scaled(w):
        nrm = jnp.sqrt(jnp.sum(w * w, axis=-1, keepdims=True))
        return w * jnp.where(nrm > 1.0, 1.0 / (nrm + 1e-7), 1.0)

    o3_ref[...] = scaled(w3_ref[...])

    @pl.when(j == 0)
    def _():
        o1_ref[...] = scaled(w1_ref[...])
        o2_ref[...] = scaled(w2_ref[...])


_prep = pl.pallas_call(
    _prep_body,
    grid=(GRID,),
    in_specs=[
        pl.BlockSpec((IDXB, LANES), lambda j: (j, 0)),
        pl.BlockSpec((V1, D), lambda j: (0, 0)),
        pl.BlockSpec((V2, D), lambda j: (0, 0)),
        pl.BlockSpec((W3B, D), lambda j: (j, 0)),
    ],
    out_specs=[
        pl.BlockSpec((IDXB, LANES), lambda j: (j, 0)),
        pl.BlockSpec((IDXB, LANES), lambda j: (j, 0)),
        pl.BlockSpec((IDXB, LANES), lambda j: (j, 0)),
        pl.BlockSpec((V1, D), lambda j: (0, 0)),
        pl.BlockSpec((V2, D), lambda j: (0, 0)),
        pl.BlockSpec((W3B, D), lambda j: (j, 0)),
    ],
    out_shape=[
        jax.ShapeDtypeStruct((NROWS, LANES), jnp.int32),
        jax.ShapeDtypeStruct((NROWS, LANES), jnp.int32),
        jax.ShapeDtypeStruct((NROWS, LANES), jnp.int32),
        jax.ShapeDtypeStruct((V1, D), jnp.float32),
        jax.ShapeDtypeStruct((V2, D), jnp.float32),
        jax.ShapeDtypeStruct((V3, D), jnp.float32),
    ],
)


@functools.partial(
    pl.kernel,
    out_type=jax.ShapeDtypeStruct((N, OUT_D), jnp.float32),
    mesh=plsc.VectorSubcoreMesh(core_axis_name="c", subcore_axis_name="s",
                                num_cores=NC, num_subcores=NS),
    compiler_params=pltpu.CompilerParams(use_tc_tiling_on_sc=False),
    scratch_types=[
        pltpu.VMEM((RPW, LANES), jnp.int32),
        pltpu.VMEM((RPW, LANES), jnp.int32),
        pltpu.VMEM((RPW, LANES), jnp.int32),
        pltpu.VMEM((3, LANES, D), jnp.float32),
        pltpu.SemaphoreType.DMA,
        pltpu.SemaphoreType.DMA,
    ],
)
def _gather(i1_hbm, i2_hbm, i3_hbm, w1_hbm, w2_hbm, w3_hbm, out_hbm,
            iv1, iv2, iv3, bufs, gsem, wsem):
    wid = lax.axis_index("s") * NC + lax.axis_index("c")
    rbase = wid * RPW

    pltpu.sync_copy(i1_hbm.at[wid], iv1)
    pltpu.sync_copy(i2_hbm.at[wid], iv2)
    pltpu.sync_copy(i3_hbm.at[wid], iv3)

    def gathers(j):
        return (
            pltpu.make_async_copy(w1_hbm.at[iv1.at[j]], bufs.at[0], gsem),
            pltpu.make_async_copy(w2_hbm.at[iv2.at[j]], bufs.at[1], gsem),
            pltpu.make_async_copy(w3_hbm.at[iv3.at[j]], bufs.at[2], gsem),
        )

    def writes(j):
        ob = (rbase + j) * LANES
        return tuple(
            pltpu.make_async_copy(
                bufs.at[t],
                out_hbm.at[pl.ds(ob, LANES), pl.ds(t * D, D)],
                wsem)
            for t in range(3)
        )

    def body(j, carry):
        g = gathers(j)
        for c in g:
            c.start()
        for c in g:
            c.wait()
        w = writes(j)
        for c in w:
            c.start()
        for c in w:
            c.wait()
        return carry

    lax.fori_loop(0, RPW, body, 0)


def kernel(idx, W1, W2, W3):
    idxr = idx.reshape(NROWS, LANES)
    i1, i2, i3, w1s, w2s, w3s = _prep(idxr, W1, W2, W3)
    i1 = i1.reshape(NW, RPW, LANES)
    i2 = i2.reshape(NW, RPW, LANES)
    i3 = i3.reshape(NW, RPW, LANES)
    out = _gather(i1, i2, i3, w1s, w2s, w3s)
    return out.reshape(BATCH, HIST, OUT_D)
